# baseline (speedup 1.0000x reference)
import jax
import jax.numpy as jnp
from jax import lax
from jax.experimental import pallas as pl
from jax.experimental.pallas import tpu as pltpu
from jax.experimental.pallas import tpu_sc as plsc

N = 10000
E = 160000
D_IN = 256
D_OUT = 256
H = 128
NC = 2
NS = 16
EDGES_PER_TILE = E // NS
CHUNK = 80
NCHUNK = EDGES_PER_TILE // CHUNK
NSTEP = 128
UNROLL = 8
NITER = NSTEP // UNROLL
ROWS_PER_TILE = 640
LAST_ROWS = N - (NS - 1) * ROWS_PER_TILE
MM_BLK = 2000


def _linear_kernel(x_ref, w_ref, b_ref, o_ref):
    acc = lax.dot_general(
        x_ref[...], w_ref[...],
        dimension_numbers=(((1,), (1,)), ((), ())),
        preferred_element_type=jnp.float32,
    )
    o_ref[...] = acc + b_ref[0]


def _linear(x, W, b2):
    grid = (NC, N // MM_BLK)
    return pl.pallas_call(
        _linear_kernel,
        grid=grid,
        in_specs=[
            pl.BlockSpec((MM_BLK, D_IN), lambda c, r: (r, 0)),
            pl.BlockSpec((H, D_IN), lambda c, r: (c, 0)),
            pl.BlockSpec((1, 1, H), lambda c, r: (c, 0, 0)),
        ],
        out_specs=pl.BlockSpec((MM_BLK, H), lambda c, r: (c * (N // MM_BLK) + r, 0)),
        out_shape=jax.ShapeDtypeStruct((NC * N, H), jnp.float32),
    )(x, W, b2)


def _spmm_body(support_hbm, col_hbm, row_hbm, val_hbm, zeros_hbm, out_hbm,
               idx_v, ridx_v, vals_v, rows_v, acc, csem, gsem, ssem):
    c = lax.axis_index("c")
    s = lax.axis_index("s")

    @pl.when(s < NS - 1)
    def _():
        pltpu.sync_copy(zeros_hbm.at[pl.ds(0, ROWS_PER_TILE)],
                        acc.at[pl.ds(s * ROWS_PER_TILE, ROWS_PER_TILE)])

    @pl.when(s == NS - 1)
    def _():
        pltpu.sync_copy(zeros_hbm.at[pl.ds(0, LAST_ROWS)],
                        acc.at[pl.ds((NS - 1) * ROWS_PER_TILE, LAST_ROWS)])

    plsc.subcore_barrier()

    ebase = s * EDGES_PER_TILE
    coff = c * N

    def chunk_off(cg):
        return ebase + jnp.minimum(cg, NCHUNK - 1) * CHUNK

    def stage0_start(cg, q, r):
        off = chunk_off(cg)
        pltpu.async_copy(col_hbm.at[pl.ds(off, CHUNK)], idx_v.at[q], csem.at[q])
        pltpu.async_copy(row_hbm.at[pl.ds(off, CHUNK)], ridx_v.at[r], csem.at[q])
        pltpu.async_copy(val_hbm.at[pl.ds(off, CHUNK)], vals_v.at[q], csem.at[q])

    def stage0_wait(cg, q, r):
        pltpu.make_async_copy(col_hbm.at[pl.ds(0, CHUNK)], idx_v.at[q], csem.at[q]).wait()
        pltpu.make_async_copy(row_hbm.at[pl.ds(0, CHUNK)], ridx_v.at[r], csem.at[q]).wait()
        pltpu.make_async_copy(val_hbm.at[pl.ds(0, CHUNK)], vals_v.at[q], csem.at[q]).wait()

        @pl.when(cg >= NCHUNK)
        def _():
            for i in range(CHUNK // 16):
                sl = pl.ds(i * 16, 16)
                ridx_v[r, sl] = jnp.zeros((16,), jnp.int32) + (N + s)

    def gather_start(q):
        for i in range(CHUNK // 16):
            sl = pl.ds(i * 16, 16)
            idx_v[q, sl] = idx_v[q, sl] + coff
        pltpu.async_copy(support_hbm.at[idx_v.at[q]], rows_v.at[q], gsem.at[q])

    def gather_wait(q):
        pltpu.make_async_copy(support_hbm.at[idx_v.at[q]], rows_v.at[q], gsem.at[q]).wait()

    def scatter_start(q, r):
        pltpu.async_copy(rows_v.at[q], acc.at[ridx_v.at[r]], ssem.at[q], add=True)

    def scatter_wait(q, r):
        pltpu.make_async_copy(rows_v.at[q], acc.at[ridx_v.at[r]], ssem.at[q]).wait()

    def multiply(q):
        def g16_body(i, _):
            vv = vals_v[q, pl.ds(i * 16, 16)]
            for lane in range(16):
                e = i * 16 + lane
                v = vv[lane]
                for j in range(H // 16):
                    sl = pl.ds(j * 16, 16)
                    rows_v[q, e, sl] = rows_v[q, e, sl] * v
            return 0

        lax.fori_loop(0, CHUNK // 16, g16_body, 0)

    for q in range(3):
        stage0_start(q, q, q)
    stage0_wait(0, 0, 0)
    gather_start(0)
    stage0_wait(1, 1, 1)
    gather_start(1)

    def step(cg, k, it):
        q, q2, q3 = k & 3, (k + 2) & 3, (k + 3) & 3
        r2, r3 = (k + 2) & 7, (k + 3) & 7
        rm2, r = (k + 6) & 7, k & 7

        gather_wait(q)

        def retire():
            scatter_wait(q2, rm2)
        if k >= 2:
            retire()
        else:
            pl.when(it >= 1)(retire)

        def next_gather():
            stage0_wait(cg + 2, q2, r2)
            gather_start(q2)
        if k < 6:
            next_gather()
        else:
            pl.when(it < NITER - 1)(next_gather)

        multiply(q)
        scatter_start(q, r)

        def refill():
            stage0_start(cg + 3, q3, r3)
        if k < 5:
            refill()
        else:
            pl.when(it < NITER - 1)(refill)

    def loop_body(it, _):
        for k in range(UNROLL):
            step(it * UNROLL + k, k, it)
        return 0

    lax.fori_loop(0, NITER, loop_body, 0)
    scatter_wait(2, 6)
    scatter_wait(3, 7)

    plsc.subcore_barrier()
    cstart = pl.multiple_of(c * H, H)

    @pl.when(s < NS - 1)
    def _():
        pltpu.sync_copy(
            acc.at[pl.ds(s * ROWS_PER_TILE, ROWS_PER_TILE)],
            out_hbm.at[pl.ds(s * ROWS_PER_TILE, ROWS_PER_TILE), pl.ds(cstart, H)],
        )

    @pl.when(s == NS - 1)
    def _():
        pltpu.sync_copy(
            acc.at[pl.ds((NS - 1) * ROWS_PER_TILE, LAST_ROWS)],
            out_hbm.at[pl.ds((NS - 1) * ROWS_PER_TILE, LAST_ROWS), pl.ds(cstart, H)],
        )


@jax.jit
def _spmm(support, col, row, vals, zeros):
    mesh = plsc.VectorSubcoreMesh(core_axis_name="c", subcore_axis_name="s")
    return pl.kernel(
        _spmm_body,
        out_type=jax.ShapeDtypeStruct((N, D_OUT), jnp.float32),
        mesh=mesh,
        scratch_types=[
            pltpu.VMEM((4, CHUNK), jnp.int32),
            pltpu.VMEM((8, CHUNK), jnp.int32),
            pltpu.VMEM((4, CHUNK), jnp.float32),
            pltpu.VMEM((4, CHUNK, H), jnp.float32),
            pltpu.VMEM_SHARED((N + NS, H), jnp.float32),
            pltpu.SemaphoreType.DMA((4,)),
            pltpu.SemaphoreType.DMA((4,)),
            pltpu.SemaphoreType.DMA((4,)),
        ],
    )(support, col, row, vals, zeros)


@jax.jit
def kernel(input, adj_indices, adj_values, W, b):
    support = _linear(input, W, b.reshape(NC, 1, H))
    zeros = jnp.zeros((ROWS_PER_TILE, H), jnp.float32)
    return _spmm(support, adj_indices[1], adj_indices[0], adj_values, zeros)


# speedup vs baseline: 1.0378x; 1.0378x over previous
import jax
import jax.numpy as jnp
from jax import lax
from jax.experimental import pallas as pl
from jax.experimental.pallas import tpu as pltpu
from jax.experimental.pallas import tpu_sc as plsc

N = 10000
E = 160000
D_IN = 256
D_OUT = 256
H = 128
NC = 2
NS = 16
EDGES_PER_TILE = E // NS
CHUNK = 88
NCHUNK = 114
LAST_OFF = EDGES_PER_TILE - CHUNK
DUP = EDGES_PER_TILE - (NCHUNK - 1) * CHUNK
NSTEP = 120
UNROLL = 8
NITER = NSTEP // UNROLL
ROWS_PER_TILE = 640
LAST_ROWS = N - (NS - 1) * ROWS_PER_TILE
MM_BLK = 2000


def _linear_kernel(x_ref, w_ref, b_ref, o_ref):
    acc = lax.dot_general(
        x_ref[...], w_ref[...],
        dimension_numbers=(((1,), (1,)), ((), ())),
        preferred_element_type=jnp.float32,
    )
    o_ref[...] = acc + b_ref[0]


def _linear(x, W, b2):
    grid = (NC, N // MM_BLK)
    return pl.pallas_call(
        _linear_kernel,
        grid=grid,
        in_specs=[
            pl.BlockSpec((MM_BLK, D_IN), lambda c, r: (r, 0)),
            pl.BlockSpec((H, D_IN), lambda c, r: (c, 0)),
            pl.BlockSpec((1, 1, H), lambda c, r: (c, 0, 0)),
        ],
        out_specs=pl.BlockSpec((MM_BLK, H), lambda c, r: (c * (N // MM_BLK) + r, 0)),
        out_shape=jax.ShapeDtypeStruct((NC * N, H), jnp.float32),
    )(x, W, b2)


def _spmm_body(support_hbm, col_hbm, row_hbm, val_hbm, zeros_hbm, out_hbm,
               idx_v, ridx_v, vals_v, rows_v, acc, csem, gsem, ssem):
    c = lax.axis_index("c")
    s = lax.axis_index("s")

    @pl.when(s < NS - 1)
    def _():
        pltpu.sync_copy(zeros_hbm.at[pl.ds(0, ROWS_PER_TILE)],
                        acc.at[pl.ds(s * ROWS_PER_TILE, ROWS_PER_TILE)])

    @pl.when(s == NS - 1)
    def _():
        pltpu.sync_copy(zeros_hbm.at[pl.ds(0, LAST_ROWS)],
                        acc.at[pl.ds((NS - 1) * ROWS_PER_TILE, LAST_ROWS)])

    plsc.subcore_barrier()

    ebase = s * EDGES_PER_TILE
    coff = c * N

    def chunk_off(cg):
        return ebase + jnp.minimum(cg * CHUNK, LAST_OFF)

    def stage0_start(cg, q, r):
        off = chunk_off(cg)
        pltpu.async_copy(col_hbm.at[pl.ds(off, CHUNK)], idx_v.at[q], csem.at[q])
        pltpu.async_copy(row_hbm.at[pl.ds(off, CHUNK)], ridx_v.at[r], csem.at[q])
        pltpu.async_copy(val_hbm.at[pl.ds(off, CHUNK)], vals_v.at[q], csem.at[q])

    def stage0_wait(cg, q, r):
        pltpu.make_async_copy(col_hbm.at[pl.ds(0, CHUNK)], idx_v.at[q], csem.at[q]).wait()
        pltpu.make_async_copy(row_hbm.at[pl.ds(0, CHUNK)], ridx_v.at[r], csem.at[q]).wait()
        pltpu.make_async_copy(val_hbm.at[pl.ds(0, CHUNK)], vals_v.at[q], csem.at[q]).wait()

        @pl.when(cg == NCHUNK - 1)
        def _():
            for i in range((CHUNK - DUP) // 16):
                vals_v[q, pl.ds(i * 16, 16)] = jnp.zeros((16,), jnp.float32)

        @pl.when(cg >= NCHUNK)
        def _():
            for i in range(CHUNK // 16):
                vals_v[q, pl.ds(i * 16, 16)] = jnp.zeros((16,), jnp.float32)

    def gather_start(q):
        for i in range(CHUNK // 16):
            sl = pl.ds(i * 16, 16)
            idx_v[q, sl] = idx_v[q, sl] + coff
        pltpu.async_copy(support_hbm.at[idx_v.at[q]], rows_v.at[q], gsem.at[q])

    def gather_wait(q):
        pltpu.make_async_copy(support_hbm.at[idx_v.at[q]], rows_v.at[q], gsem.at[q]).wait()

    def scatter_start(q, r):
        pltpu.async_copy(rows_v.at[q], acc.at[ridx_v.at[r]], ssem.at[q], add=True)

    def scatter_wait(q, r):
        pltpu.make_async_copy(rows_v.at[q], acc.at[ridx_v.at[r]], ssem.at[q]).wait()

    def multiply(q):
        def g16_body(i, _):
            vv = vals_v[q, pl.ds(i * 16, 16)]
            for lane in range(16):
                e = i * 16 + lane
                v = vv[lane]
                for j in range(H // 16):
                    sl = pl.ds(j * 16, 16)
                    rows_v[q, e, sl] = rows_v[q, e, sl] * v
            return 0

        lax.fori_loop(0, CHUNK // 16, g16_body, 0)

    for q in range(3):
        stage0_start(q, q, q)
    stage0_wait(0, 0, 0)
    gather_start(0)
    stage0_wait(1, 1, 1)
    gather_start(1)

    def step(cg, k, it):
        q, q2, q3 = k & 3, (k + 2) & 3, (k + 3) & 3
        r2, r3 = (k + 2) & 7, (k + 3) & 7
        rm2, r = (k + 6) & 7, k & 7

        gather_wait(q)

        def retire():
            scatter_wait(q2, rm2)
        if k >= 2:
            retire()
        else:
            pl.when(it >= 1)(retire)

        def next_gather():
            stage0_wait(cg + 2, q2, r2)
            gather_start(q2)
        if k < 6:
            next_gather()
        else:
            pl.when(it < NITER - 1)(next_gather)

        multiply(q)
        scatter_start(q, r)

        def refill():
            stage0_start(cg + 3, q3, r3)
        if k < 5:
            refill()
        else:
            pl.when(it < NITER - 1)(refill)

    def loop_body(it, _):
        for k in range(UNROLL):
            step(it * UNROLL + k, k, it)
        return 0

    lax.fori_loop(0, NITER, loop_body, 0)
    scatter_wait(2, 6)
    scatter_wait(3, 7)

    plsc.subcore_barrier()
    cstart = pl.multiple_of(c * H, H)

    @pl.when(s < NS - 1)
    def _():
        pltpu.sync_copy(
            acc.at[pl.ds(s * ROWS_PER_TILE, ROWS_PER_TILE)],
            out_hbm.at[pl.ds(s * ROWS_PER_TILE, ROWS_PER_TILE), pl.ds(cstart, H)],
        )

    @pl.when(s == NS - 1)
    def _():
        pltpu.sync_copy(
            acc.at[pl.ds((NS - 1) * ROWS_PER_TILE, LAST_ROWS)],
            out_hbm.at[pl.ds((NS - 1) * ROWS_PER_TILE, LAST_ROWS), pl.ds(cstart, H)],
        )


@jax.jit
def _spmm(support, col, row, vals, zeros):
    mesh = plsc.VectorSubcoreMesh(core_axis_name="c", subcore_axis_name="s")
    return pl.kernel(
        _spmm_body,
        out_type=jax.ShapeDtypeStruct((N, D_OUT), jnp.float32),
        mesh=mesh,
        scratch_types=[
            pltpu.VMEM((4, CHUNK), jnp.int32),
            pltpu.VMEM((8, CHUNK), jnp.int32),
            pltpu.VMEM((4, CHUNK), jnp.float32),
            pltpu.VMEM((4, CHUNK, H), jnp.float32),
            pltpu.VMEM_SHARED((N, H), jnp.float32),
            pltpu.SemaphoreType.DMA((4,)),
            pltpu.SemaphoreType.DMA((4,)),
            pltpu.SemaphoreType.DMA((4,)),
        ],
    )(support, col, row, vals, zeros)


@jax.jit
def kernel(input, adj_indices, adj_values, W, b):
    support = _linear(input, W, b.reshape(NC, 1, H))
    zeros = jnp.zeros((ROWS_PER_TILE, H), jnp.float32)
    return _spmm(support, adj_indices[1], adj_indices[0], adj_values, zeros)
